# baseline (speedup 1.0000x reference)
import functools

import jax
import jax.numpy as jnp
from jax import lax
from jax.experimental import pallas as pl
from jax.experimental.pallas import tpu as pltpu
from jax.experimental.pallas import tpu_sc as plsc


@functools.lru_cache(maxsize=None)
def _build(B, H, W):
    info = plsc.get_sparse_core_info()
    NC, NS, L = info.num_cores, info.num_subcores, info.num_lanes
    NW = NC * NS
    assert B % NW == 0
    bpw = B // NW
    C = 2048
    assert bpw % C == 0
    n_chunks = bpw // C

    mesh = plsc.VectorSubcoreMesh(core_axis_name="c", subcore_axis_name="s")

    @functools.partial(
        pl.kernel,
        mesh=mesh,
        out_type=jax.ShapeDtypeStruct((B,), jnp.float32),
        scratch_types=[
            pltpu.VMEM((C,), jnp.float32),
            pltpu.VMEM((C,), jnp.float32),
            pltpu.VMEM((C,), jnp.int32),
            pltpu.VMEM((C,), jnp.float32),
            pltpu.VMEM((4, L), jnp.float32),
            pltpu.SemaphoreType.DMA,
        ],
    )
    def k(zx_hbm, zy_hbm, d_hbm, p_hbm, out_hbm, zxb, zyb, idxbuf, valbuf, pv, sem):
        wid = lax.axis_index("s") * NC + lax.axis_index("c")
        base = wid * bpw
        pltpu.sync_copy(p_hbm, pv)
        shift_x = pv[0]
        shift_y = pv[1]
        scale_x = pv[2]
        scale_y = pv[3]
        szx = jnp.float32(H - 1)
        szy = jnp.float32(W - 1)

        def chunk_body(c, carry):
            cb = base + c * C
            pltpu.sync_copy(zx_hbm.at[pl.ds(cb, C)], zxb)
            pltpu.sync_copy(zy_hbm.at[pl.ds(cb, C)], zyb)

            def vec_body(j, carry2):
                vx = zxb[pl.ds(j * L, L)]
                vy = zyb[pl.ds(j * L, L)]
                tx = jnp.clip((vx - shift_x) / scale_x, 0.0, 1.0)
                ty = jnp.clip((vy - shift_y) / scale_y, 0.0, 1.0)
                ix = (tx * szx).astype(jnp.int32)
                iy = (ty * szy).astype(jnp.int32)
                idxbuf[pl.ds(j * L, L)] = ix * W + iy
                return carry2

            lax.fori_loop(0, C // L, vec_body, 0)
            pltpu.async_copy(d_hbm.at[idxbuf], valbuf, sem).wait()
            pltpu.sync_copy(valbuf, out_hbm.at[pl.ds(cb, C)])
            return carry

        lax.fori_loop(0, n_chunks, chunk_body, 0)

    return k


def kernel(z, density, scale, shift, image_size):
    B = z.shape[0]
    H, W = density.shape
    L = plsc.get_sparse_core_info().num_lanes
    zx = z[:, 0]
    zy = z[:, 1]
    dflat = density.reshape(-1)
    params = jnp.concatenate(
        [
            jnp.broadcast_to(shift.reshape(2, 1), (2, L)),
            jnp.broadcast_to(scale.reshape(2, 1), (2, L)),
        ],
        axis=0,
    ).astype(jnp.float32)
    return _build(B, H, W)(zx, zy, dflat, params)


# speedup vs baseline: 1.2390x; 1.2390x over previous
import functools

import jax
import jax.numpy as jnp
from jax import lax
from jax.experimental import pallas as pl
from jax.experimental.pallas import tpu as pltpu
from jax.experimental.pallas import tpu_sc as plsc


@functools.lru_cache(maxsize=None)
def _build(B, H, W):
    info = plsc.get_sparse_core_info()
    NC, NS, L = info.num_cores, info.num_subcores, info.num_lanes
    NW = NC * NS
    assert B % NW == 0
    bpw = B // NW
    C = 4096
    assert bpw % C == 0
    n_chunks = bpw // C

    mesh = plsc.VectorSubcoreMesh(core_axis_name="c", subcore_axis_name="s")

    scratch = {}
    for b in (0, 1):
        scratch[f"zx{b}"] = pltpu.VMEM((C,), jnp.float32)
        scratch[f"zy{b}"] = pltpu.VMEM((C,), jnp.float32)
        scratch[f"idx{b}"] = pltpu.VMEM((C,), jnp.int32)
        scratch[f"val{b}"] = pltpu.VMEM((C,), jnp.float32)
        scratch[f"sem_z{b}"] = pltpu.SemaphoreType.DMA
        scratch[f"sem_g{b}"] = pltpu.SemaphoreType.DMA
        scratch[f"sem_o{b}"] = pltpu.SemaphoreType.DMA
    scratch["pv"] = pltpu.VMEM((4, L), jnp.float32)
    scratch["sem_p"] = pltpu.SemaphoreType.DMA

    @functools.partial(
        pl.kernel,
        mesh=mesh,
        out_type=jax.ShapeDtypeStruct((B,), jnp.float32),
        scratch_types=scratch,
    )
    def k(zx_hbm, zy_hbm, d_hbm, p_hbm, out_hbm, **s):
        wid = lax.axis_index("s") * NC + lax.axis_index("c")
        base = wid * bpw
        pltpu.async_copy(p_hbm, s["pv"], s["sem_p"]).wait()
        shift_x = s["pv"][0]
        shift_y = s["pv"][1]
        scale_x = s["pv"][2]
        scale_y = s["pv"][3]
        szx = jnp.float32(H - 1)
        szy = jnp.float32(W - 1)

        zbufs = [(s["zx0"], s["zy0"], s["sem_z0"]), (s["zx1"], s["zy1"], s["sem_z1"])]
        gbufs = [(s["idx0"], s["val0"], s["sem_g0"], s["sem_o0"]),
                 (s["idx1"], s["val1"], s["sem_g1"], s["sem_o1"])]

        def start_z(c):
            zx, zy, sem = zbufs[c % 2]
            cb = base + c * C
            pltpu.async_copy(zx_hbm.at[pl.ds(cb, C)], zx, sem)
            pltpu.async_copy(zy_hbm.at[pl.ds(cb, C)], zy, sem)

        def wait_z(c):
            zx, zy, sem = zbufs[c % 2]
            cb = base + c * C
            pltpu.make_async_copy(zx_hbm.at[pl.ds(cb, C)], zx, sem).wait()
            pltpu.make_async_copy(zy_hbm.at[pl.ds(cb, C)], zy, sem).wait()

        def compute_idx(c):
            zx, zy, _ = zbufs[c % 2]
            idx = gbufs[c % 2][0]

            def vec_body(j, carry):
                vx = zx[pl.ds(j * L, L)]
                vy = zy[pl.ds(j * L, L)]
                tx = jnp.clip((vx - shift_x) / scale_x, 0.0, 1.0)
                ty = jnp.clip((vy - shift_y) / scale_y, 0.0, 1.0)
                ix = (tx * szx).astype(jnp.int32)
                iy = (ty * szy).astype(jnp.int32)
                idx[pl.ds(j * L, L)] = ix * W + iy
                return carry

            lax.fori_loop(0, C // L, vec_body, 0, unroll=4)

        def start_gather(c):
            idx, val, sem, _ = gbufs[c % 2]
            pltpu.async_copy(d_hbm.at[idx], val, sem)

        def wait_gather(c):
            idx, val, sem, _ = gbufs[c % 2]
            pltpu.make_async_copy(d_hbm.at[idx], val, sem).wait()

        def start_out(c):
            _, val, _, sem = gbufs[c % 2]
            cb = base + c * C
            pltpu.async_copy(val, out_hbm.at[pl.ds(cb, C)], sem)

        def wait_out(c):
            _, val, _, sem = gbufs[c % 2]
            cb = base + c * C
            pltpu.make_async_copy(val, out_hbm.at[pl.ds(cb, C)], sem).wait()

        start_z(0)
        wait_z(0)
        compute_idx(0)
        start_gather(0)
        start_z(1)
        for c in range(1, n_chunks):
            wait_z(c)
            compute_idx(c)
            wait_gather(c - 1)
            start_gather(c)
            start_out(c - 1)
            if c + 1 < n_chunks:
                start_z(c + 1)
            if c >= 2:
                wait_out(c - 2)
        wait_gather(n_chunks - 1)
        start_out(n_chunks - 1)
        if n_chunks >= 2:
            wait_out(n_chunks - 2)
        wait_out(n_chunks - 1)

    return k


def kernel(z, density, scale, shift, image_size):
    B = z.shape[0]
    H, W = density.shape
    L = plsc.get_sparse_core_info().num_lanes
    zx = z[:, 0]
    zy = z[:, 1]
    dflat = density.reshape(-1)
    params = jnp.concatenate(
        [
            jnp.broadcast_to(shift.reshape(2, 1), (2, L)),
            jnp.broadcast_to(scale.reshape(2, 1), (2, L)),
        ],
        axis=0,
    ).astype(jnp.float32)
    return _build(B, H, W)(zx, zy, dflat, params)


# speedup vs baseline: 1.2628x; 1.0192x over previous
import functools

import jax
import jax.numpy as jnp
from jax import lax
from jax.experimental import pallas as pl
from jax.experimental.pallas import tpu as pltpu
from jax.experimental.pallas import tpu_sc as plsc


@functools.lru_cache(maxsize=None)
def _build(B, H, W):
    info = plsc.get_sparse_core_info()
    NC, NS, L = info.num_cores, info.num_subcores, info.num_lanes
    NW = NC * NS
    assert B % NW == 0
    bpw = B // NW
    C = 4096
    assert bpw % C == 0
    n_chunks = bpw // C

    mesh = plsc.VectorSubcoreMesh(core_axis_name="c", subcore_axis_name="s")

    NB = 4
    scratch = {}
    for b in range(NB):
        scratch[f"zx{b}"] = pltpu.VMEM((C,), jnp.float32)
        scratch[f"zy{b}"] = pltpu.VMEM((C,), jnp.float32)
        scratch[f"idx{b}"] = pltpu.VMEM((C,), jnp.int32)
        scratch[f"val{b}"] = pltpu.VMEM((C,), jnp.float32)
        scratch[f"sem_z{b}"] = pltpu.SemaphoreType.DMA
        scratch[f"sem_g{b}"] = pltpu.SemaphoreType.DMA
        scratch[f"sem_o{b}"] = pltpu.SemaphoreType.DMA
    scratch["pv"] = pltpu.VMEM((4, L), jnp.float32)
    scratch["sem_p"] = pltpu.SemaphoreType.DMA

    @functools.partial(
        pl.kernel,
        mesh=mesh,
        out_type=jax.ShapeDtypeStruct((B,), jnp.float32),
        scratch_types=scratch,
    )
    def k(zx_hbm, zy_hbm, d_hbm, p_hbm, out_hbm, **s):
        wid = lax.axis_index("s") * NC + lax.axis_index("c")
        base = wid * bpw
        pltpu.async_copy(p_hbm, s["pv"], s["sem_p"]).wait()
        shift_x = s["pv"][0]
        shift_y = s["pv"][1]
        scale_x = s["pv"][2]
        scale_y = s["pv"][3]
        szx = jnp.float32(H - 1)
        szy = jnp.float32(W - 1)

        zbufs = [(s[f"zx{b}"], s[f"zy{b}"], s[f"sem_z{b}"]) for b in range(NB)]
        gbufs = [(s[f"idx{b}"], s[f"val{b}"], s[f"sem_g{b}"], s[f"sem_o{b}"])
                 for b in range(NB)]

        def start_z(c):
            zx, zy, sem = zbufs[c % NB]
            cb = base + c * C
            pltpu.async_copy(zx_hbm.at[pl.ds(cb, C)], zx, sem)
            pltpu.async_copy(zy_hbm.at[pl.ds(cb, C)], zy, sem)

        def wait_z(c):
            zx, zy, sem = zbufs[c % NB]
            cb = base + c * C
            pltpu.make_async_copy(zx_hbm.at[pl.ds(cb, C)], zx, sem).wait()
            pltpu.make_async_copy(zy_hbm.at[pl.ds(cb, C)], zy, sem).wait()

        def compute_idx(c):
            zx, zy, _ = zbufs[c % NB]
            idx = gbufs[c % NB][0]

            def vec_body(j, carry):
                vx = zx[pl.ds(j * L, L)]
                vy = zy[pl.ds(j * L, L)]
                tx = jnp.clip((vx - shift_x) / scale_x, 0.0, 1.0)
                ty = jnp.clip((vy - shift_y) / scale_y, 0.0, 1.0)
                ix = (tx * szx).astype(jnp.int32)
                iy = (ty * szy).astype(jnp.int32)
                idx[pl.ds(j * L, L)] = ix * W + iy
                return carry

            lax.fori_loop(0, C // L, vec_body, 0, unroll=4)

        def start_gather(c):
            idx, val, sem, _ = gbufs[c % NB]
            pltpu.async_copy(d_hbm.at[idx], val, sem)

        def wait_gather(c):
            idx, val, sem, _ = gbufs[c % NB]
            pltpu.make_async_copy(d_hbm.at[idx], val, sem).wait()

        def start_out(c):
            _, val, _, sem = gbufs[c % NB]
            cb = base + c * C
            pltpu.async_copy(val, out_hbm.at[pl.ds(cb, C)], sem)

        def wait_out(c):
            _, val, _, sem = gbufs[c % NB]
            cb = base + c * C
            pltpu.make_async_copy(val, out_hbm.at[pl.ds(cb, C)], sem).wait()

        start_z(0)
        start_z(1)
        wait_z(0)
        compute_idx(0)
        start_gather(0)
        wait_z(1)
        compute_idx(1)
        start_gather(1)
        for c in range(2, n_chunks):
            start_z(c)
            wait_z(c)
            compute_idx(c)
            wait_gather(c - 2)
            start_out(c - 2)
            if c >= 4:
                wait_out(c - 4)
            start_gather(c)
        for c in range(max(0, n_chunks - 2), n_chunks):
            wait_gather(c)
            start_out(c)
        for c in range(max(0, n_chunks - 4), n_chunks):
            wait_out(c)

    return k


def kernel(z, density, scale, shift, image_size):
    B = z.shape[0]
    H, W = density.shape
    L = plsc.get_sparse_core_info().num_lanes
    zx = z[:, 0]
    zy = z[:, 1]
    dflat = density.reshape(-1)
    params = jnp.concatenate(
        [
            jnp.broadcast_to(shift.reshape(2, 1), (2, L)),
            jnp.broadcast_to(scale.reshape(2, 1), (2, L)),
        ],
        axis=0,
    ).astype(jnp.float32)
    return _build(B, H, W)(zx, zy, dflat, params)
